# Initial kernel scaffold; baseline (speedup 1.0000x reference)
import jax
import jax.numpy as jnp
from jax.experimental import pallas as pl


def kernel(x, edge_index, batch, W1, b1, W2, b2):
    raise NotImplementedError("write your pallas kernel here")


# speedup vs baseline: 14.5129x; 14.5129x over previous
import functools

import jax
import jax.numpy as jnp
from jax import lax
from jax.experimental import pallas as pl
from jax.experimental.pallas import tpu as pltpu
from jax.experimental.pallas import tpu_sc as plsc

NC = 2
NS = 16
K = 80


def _mesh():
    return plsc.VectorSubcoreMesh(core_axis_name="c", subcore_axis_name="s")


def _make_deg_kernel(E, NP):
    per_w = E // (NC * NS)
    chunks = per_w // K
    span = NP // NS

    @functools.partial(
        pl.kernel,
        mesh=_mesh(),
        out_type=jax.ShapeDtypeStruct((NC, NP), jnp.float32),
        scratch_types=[
            pltpu.VMEM((K,), jnp.int32),
            pltpu.VMEM((K,), jnp.float32),
            pltpu.VMEM((span,), jnp.float32),
            pltpu.VMEM_SHARED((NP,), jnp.float32),
        ],
    )
    def deg_kernel(dst_hbm, out_hbm, idx_v, ones_v, zbuf, deg_sh):
        c = lax.axis_index("c")
        s = lax.axis_index("s")
        for i in range(K // 16):
            ones_v[pl.ds(16 * i, 16)] = jnp.ones((16,), jnp.float32)
        for i in range(span // 16):
            zbuf[pl.ds(16 * i, 16)] = jnp.zeros((16,), jnp.float32)
        pltpu.sync_copy(zbuf, deg_sh.at[pl.ds(s * span, span)])
        plsc.subcore_barrier()
        base0 = (c * NS + s) * per_w

        def chunk(j, carry):
            off = pl.multiple_of(base0 + j * K, 8)
            pltpu.sync_copy(dst_hbm.at[pl.ds(off, K)], idx_v)
            pltpu.sync_copy(ones_v, deg_sh.at[idx_v], add=True)
            return carry

        lax.fori_loop(0, chunks, chunk, 0)
        plsc.subcore_barrier()
        pltpu.sync_copy(deg_sh.at[pl.ds(s * span, span)],
                        out_hbm.at[c, pl.ds(s * span, span)])

    return deg_kernel


def _make_prop_kernel(E, NP, D):
    per_w = E // (NC * NS)
    chunks = per_w // K
    span = NP // NS

    @functools.partial(
        pl.kernel,
        mesh=_mesh(),
        out_type=jax.ShapeDtypeStruct((NC, NP, D), jnp.float32),
        scratch_types=[
            pltpu.VMEM((K,), jnp.int32),
            pltpu.VMEM((K,), jnp.int32),
            pltpu.VMEM((K, D), jnp.float32),
            pltpu.VMEM_SHARED((NP, D), jnp.float32),
            pltpu.SemaphoreType.DMA,
        ],
    )
    def prop_kernel(src_hbm, dst_hbm, u_hbm, zeros_hbm, out_hbm,
                    sidx, didx, rows, acc_sh, sem):
        c = lax.axis_index("c")
        s = lax.axis_index("s")
        sp = pl.ds(s * span, span)
        pltpu.sync_copy(zeros_hbm.at[sp, :], acc_sh.at[sp, :])
        plsc.subcore_barrier()
        base0 = (c * NS + s) * per_w

        def chunk(j, carry):
            off = pl.multiple_of(base0 + j * K, 8)
            pltpu.sync_copy(src_hbm.at[pl.ds(off, K)], sidx)
            pltpu.sync_copy(dst_hbm.at[pl.ds(off, K)], didx)
            pltpu.async_copy(u_hbm.at[sidx], rows, sem).wait()
            pltpu.sync_copy(rows, acc_sh.at[didx], add=True)
            return carry

        lax.fori_loop(0, chunks, chunk, 0)
        plsc.subcore_barrier()
        pltpu.sync_copy(acc_sh.at[sp, :], out_hbm.at[c, sp, :])

    return prop_kernel


def _make_thin_prop_kernel(E, NP):
    per_w = E // (NC * NS)
    chunks = per_w // K
    span = NP // NS

    @functools.partial(
        pl.kernel,
        mesh=_mesh(),
        out_type=jax.ShapeDtypeStruct((NC, 2, NP), jnp.float32),
        scratch_types=[
            pltpu.VMEM((K,), jnp.int32),
            pltpu.VMEM((K,), jnp.int32),
            pltpu.VMEM((K,), jnp.float32),
            pltpu.VMEM((K,), jnp.float32),
            pltpu.VMEM((span,), jnp.float32),
            pltpu.VMEM_SHARED((NP,), jnp.float32),
            pltpu.VMEM_SHARED((NP,), jnp.float32),
            pltpu.SemaphoreType.DMA,
        ],
    )
    def thin_prop(src_hbm, dst_hbm, u20_hbm, u21_hbm, out_hbm,
                  sidx, didx, vals0, vals1, zbuf, acc0_sh, acc1_sh, sem):
        c = lax.axis_index("c")
        s = lax.axis_index("s")
        sp = pl.ds(s * span, span)
        for i in range(span // 16):
            zbuf[pl.ds(16 * i, 16)] = jnp.zeros((16,), jnp.float32)
        pltpu.sync_copy(zbuf, acc0_sh.at[sp])
        pltpu.sync_copy(zbuf, acc1_sh.at[sp])
        plsc.subcore_barrier()
        base0 = (c * NS + s) * per_w

        def chunk(j, carry):
            off = pl.multiple_of(base0 + j * K, 8)
            pltpu.sync_copy(src_hbm.at[pl.ds(off, K)], sidx)
            pltpu.sync_copy(dst_hbm.at[pl.ds(off, K)], didx)
            pltpu.async_copy(u20_hbm.at[sidx], vals0, sem).wait()
            pltpu.async_copy(u21_hbm.at[sidx], vals1, sem).wait()
            pltpu.sync_copy(vals0, acc0_sh.at[didx], add=True)
            pltpu.sync_copy(vals1, acc1_sh.at[didx], add=True)
            return carry

        lax.fori_loop(0, chunks, chunk, 0)
        plsc.subcore_barrier()
        pltpu.sync_copy(acc0_sh.at[sp], out_hbm.at[c, 0, sp])
        pltpu.sync_copy(acc1_sh.at[sp], out_hbm.at[c, 1, sp])

    return thin_prop


def _deg_to_dinv(deg_ref, dd_ref):
    d = deg_ref[...]
    deg = d[0:1, :] + d[1:2, :] + 1.0
    dinv = lax.rsqrt(deg)
    dd_ref[0:1, :] = dinv
    dd_ref[1:2, :] = dinv * dinv


def _scale_rows(x_ref, dinv_ref, u_ref):
    u_ref[...] = x_ref[...] * dinv_ref[...]


def _dense_block(acc0_ref, acc1_ref, x_ref, dinv_ref, dinv2_ref,
                 w1_ref, b1_ref, w2_ref, g_ref, u2_ref):
    y = dinv_ref[...] * (acc0_ref[...] + acc1_ref[...]) + dinv2_ref[...] * x_ref[...]
    h = jnp.dot(y, w1_ref[...], preferred_element_type=jnp.float32) + b1_ref[...]
    h = jnp.maximum(h, 0.0)
    g = jnp.dot(h, w2_ref[...], preferred_element_type=jnp.float32)
    g_ref[...] = g
    u2_ref[...] = dinv_ref[...] * g


def _make_final(NP, NB):
    def final_body(a20_ref, a21_ref, gt_ref, dd_ref, b2_ref, batch_ref, out_ref):
        dinv = dd_ref[0:1, :]
        dinv2 = dd_ref[1:2, :]
        z0 = (dinv * (a20_ref[0:1, :] + a20_ref[1:2, :])
              + dinv2 * gt_ref[0:1, :] + b2_ref[0, 0])
        z1 = (dinv * (a21_ref[0:1, :] + a21_ref[1:2, :])
              + dinv2 * gt_ref[1:2, :] + b2_ref[0, 1])
        bt = batch_ref[...]
        gids = lax.broadcasted_iota(jnp.int32, (NB, NP), 0)
        onehot = (bt == gids).astype(jnp.float32)
        cnt = jnp.sum(onehot, axis=1, keepdims=True)
        s0 = jnp.sum(onehot * z0, axis=1, keepdims=True)
        s1 = jnp.sum(onehot * z1, axis=1, keepdims=True)
        out_ref[...] = jnp.concatenate([s0, s1], axis=1) / jnp.maximum(cnt, 1.0)

    return final_body


def kernel(x, edge_index, batch, W1, b1, W2, b2):
    N, D_IN = x.shape
    E = edge_index.shape[1]
    D_H = W1.shape[1]
    D_OUT = W2.shape[1]
    NB = 16

    NP = ((N + 511) // 512) * 512
    assert E % (NC * NS * K) == 0

    src = edge_index[0]
    dst = edge_index[1]
    xp = jnp.pad(x, ((0, NP - N), (0, 0)))
    batchp = jnp.pad(batch, (0, NP - N), constant_values=NB).reshape(1, NP)

    deg_p = _make_deg_kernel(E, NP)(dst)

    dd = pl.pallas_call(
        _deg_to_dinv,
        out_shape=jax.ShapeDtypeStruct((2, NP), jnp.float32),
    )(deg_p)
    dinv_c = dd[0].reshape(NP, 1)
    dinv2_c = dd[1].reshape(NP, 1)

    RB = 1024
    nblk = NP // RB
    u = pl.pallas_call(
        _scale_rows,
        grid=(nblk,),
        in_specs=[
            pl.BlockSpec((RB, D_IN), lambda i: (i, 0)),
            pl.BlockSpec((RB, 1), lambda i: (i, 0)),
        ],
        out_specs=pl.BlockSpec((RB, D_IN), lambda i: (i, 0)),
        out_shape=jax.ShapeDtypeStruct((NP, D_IN), jnp.float32),
    )(xp, dinv_c)

    zeros_wide = jnp.zeros((NP, D_IN), jnp.float32)
    acc = _make_prop_kernel(E, NP, D_IN)(src, dst, u, zeros_wide)

    g, u2 = pl.pallas_call(
        _dense_block,
        grid=(nblk,),
        in_specs=[
            pl.BlockSpec((RB, D_IN), lambda i: (i, 0)),
            pl.BlockSpec((RB, D_IN), lambda i: (i, 0)),
            pl.BlockSpec((RB, D_IN), lambda i: (i, 0)),
            pl.BlockSpec((RB, 1), lambda i: (i, 0)),
            pl.BlockSpec((RB, 1), lambda i: (i, 0)),
            pl.BlockSpec((D_IN, D_H), lambda i: (0, 0)),
            pl.BlockSpec((1, D_H), lambda i: (0, 0)),
            pl.BlockSpec((D_H, D_OUT), lambda i: (0, 0)),
        ],
        out_specs=[
            pl.BlockSpec((RB, D_OUT), lambda i: (i, 0)),
            pl.BlockSpec((RB, D_OUT), lambda i: (i, 0)),
        ],
        out_shape=[
            jax.ShapeDtypeStruct((NP, D_OUT), jnp.float32),
            jax.ShapeDtypeStruct((NP, D_OUT), jnp.float32),
        ],
    )(acc[0], acc[1], xp, dinv_c, dinv2_c, W1, b1.reshape(1, D_H), W2)

    gt = g.T
    u2t = u2.T
    acc2 = _make_thin_prop_kernel(E, NP)(src, dst, u2t[0], u2t[1])

    out = pl.pallas_call(
        _make_final(NP, NB),
        out_shape=jax.ShapeDtypeStruct((NB, D_OUT), jnp.float32),
    )(acc2[:, 0, :], acc2[:, 1, :], gt, dd, b2.reshape(1, D_OUT), batchp)
    return out


# speedup vs baseline: 15.0409x; 1.0364x over previous
import functools

import jax
import jax.numpy as jnp
from jax import lax
from jax.experimental import pallas as pl
from jax.experimental.pallas import tpu as pltpu
from jax.experimental.pallas import tpu_sc as plsc

NC = 2
NS = 16
K = 128
NBUF = 1


def _mesh():
    return plsc.VectorSubcoreMesh(core_axis_name="c", subcore_axis_name="s")


def _make_deg_kernel(CW, NP):
    span = NP // NS

    @functools.partial(
        pl.kernel,
        mesh=_mesh(),
        out_type=jax.ShapeDtypeStruct((NC, NP), jnp.float32),
        scratch_types=[
            pltpu.VMEM((CW, K), jnp.int32),
            pltpu.VMEM((K,), jnp.float32),
            pltpu.VMEM((span,), jnp.float32),
            pltpu.VMEM_SHARED((NP,), jnp.float32),
            pltpu.SemaphoreType.DMA,
        ],
    )
    def deg_kernel(dst2d_hbm, out_hbm, didx, ones_v, zbuf, deg_sh, sem):
        c = lax.axis_index("c")
        s = lax.axis_index("s")
        w = c * NS + s
        for i in range(K // 16):
            ones_v[pl.ds(16 * i, 16)] = jnp.ones((16,), jnp.float32)
        for i in range(span // 16):
            zbuf[pl.ds(16 * i, 16)] = jnp.zeros((16,), jnp.float32)
        pltpu.sync_copy(dst2d_hbm.at[pl.ds(w * CW, CW), :], didx)
        pltpu.sync_copy(zbuf, deg_sh.at[pl.ds(s * span, span)])
        plsc.subcore_barrier()

        def fire(j, carry):
            pltpu.async_copy(ones_v, deg_sh.at[didx.at[j]], sem, add=True)
            return carry

        def drain(j, carry):
            pltpu.make_async_copy(ones_v, deg_sh.at[didx.at[j]], sem).wait()
            return carry

        lax.fori_loop(0, CW, fire, 0)
        lax.fori_loop(0, CW, drain, 0)
        plsc.subcore_barrier()
        pltpu.sync_copy(deg_sh.at[pl.ds(s * span, span)],
                        out_hbm.at[c, pl.ds(s * span, span)])

    return deg_kernel


def _make_prop_kernel(CW, NP, D):
    span = NP // NS
    assert CW % NBUF == 0

    @functools.partial(
        pl.kernel,
        mesh=_mesh(),
        out_type=jax.ShapeDtypeStruct((NC, NP, D), jnp.float32),
        scratch_types=(
            [pltpu.VMEM((CW, K), jnp.int32),
             pltpu.VMEM((CW, K), jnp.int32)]
            + [pltpu.VMEM((K, D), jnp.float32)] * NBUF
            + [pltpu.VMEM_SHARED((NP, D), jnp.float32)]
            + [pltpu.SemaphoreType.DMA] * (2 * NBUF)
        ),
    )
    def prop_kernel(src2d_hbm, dst2d_hbm, u_hbm, zeros_hbm, out_hbm,
                    sidx, didx, *rest):
        rows = rest[:NBUF]
        acc_sh = rest[NBUF]
        gsem = rest[NBUF + 1:NBUF + 1 + NBUF]
        ssem = rest[NBUF + 1 + NBUF:]
        c = lax.axis_index("c")
        s = lax.axis_index("s")
        w = c * NS + s
        sp = pl.ds(s * span, span)
        pltpu.sync_copy(src2d_hbm.at[pl.ds(w * CW, CW), :], sidx)
        pltpu.sync_copy(dst2d_hbm.at[pl.ds(w * CW, CW), :], didx)
        pltpu.sync_copy(zeros_hbm.at[sp, :], acc_sh.at[sp, :])
        plsc.subcore_barrier()

        for b in range(NBUF):
            pltpu.async_copy(u_hbm.at[sidx.at[b]], rows[b], gsem[b])

        def step(t, carry):
            for b in range(NBUF):
                j = t * NBUF + b
                pltpu.make_async_copy(
                    u_hbm.at[sidx.at[j]], rows[b], gsem[b]).wait()
                pltpu.sync_copy(rows[b], acc_sh.at[didx.at[j]], add=True)

                @pl.when(j + NBUF < CW)
                def _():
                    pltpu.async_copy(
                        u_hbm.at[sidx.at[j + NBUF]], rows[b], gsem[b])
            return carry

        lax.fori_loop(0, CW // NBUF, step, 0)
        plsc.subcore_barrier()
        pltpu.sync_copy(acc_sh.at[sp, :], out_hbm.at[c, sp, :])

    return prop_kernel


def _make_thin_prop_kernel(CW, NP):
    span = NP // NS
    assert CW % NBUF == 0

    @functools.partial(
        pl.kernel,
        mesh=_mesh(),
        out_type=jax.ShapeDtypeStruct((NC, 2, NP), jnp.float32),
        scratch_types=(
            [pltpu.VMEM((CW, K), jnp.int32),
             pltpu.VMEM((CW, K), jnp.int32)]
            + [pltpu.VMEM((K,), jnp.float32)] * (2 * NBUF)
            + [pltpu.VMEM((span,), jnp.float32)]
            + [pltpu.VMEM_SHARED((NP,), jnp.float32)] * 2
            + [pltpu.SemaphoreType.DMA] * (2 * NBUF)
        ),
    )
    def thin_prop(src2d_hbm, dst2d_hbm, u20_hbm, u21_hbm, out_hbm,
                  sidx, didx, *rest):
        vals0 = rest[:NBUF]
        vals1 = rest[NBUF:2 * NBUF]
        zbuf = rest[2 * NBUF]
        acc0_sh = rest[2 * NBUF + 1]
        acc1_sh = rest[2 * NBUF + 2]
        gsem = rest[2 * NBUF + 3:2 * NBUF + 3 + NBUF]
        ssem = rest[2 * NBUF + 3 + NBUF:]
        c = lax.axis_index("c")
        s = lax.axis_index("s")
        w = c * NS + s
        sp = pl.ds(s * span, span)
        pltpu.sync_copy(src2d_hbm.at[pl.ds(w * CW, CW), :], sidx)
        pltpu.sync_copy(dst2d_hbm.at[pl.ds(w * CW, CW), :], didx)
        for i in range(span // 16):
            zbuf[pl.ds(16 * i, 16)] = jnp.zeros((16,), jnp.float32)
        pltpu.sync_copy(zbuf, acc0_sh.at[sp])
        pltpu.sync_copy(zbuf, acc1_sh.at[sp])
        plsc.subcore_barrier()

        def step(t, carry):
            @pl.when(t > 0)
            def _():
                for b in range(NBUF):
                    pltpu.make_async_copy(
                        vals0[b], acc0_sh.at[didx.at[0]], ssem[b]).wait()
                    pltpu.make_async_copy(
                        vals1[b], acc1_sh.at[didx.at[0]], ssem[b]).wait()

            gathers = []
            for b in range(NBUF):
                j = t * NBUF + b
                gathers.append(
                    pltpu.async_copy(u20_hbm.at[sidx.at[j]], vals0[b], gsem[b]))
                gathers.append(
                    pltpu.async_copy(u21_hbm.at[sidx.at[j]], vals1[b], gsem[b]))
            for b in range(NBUF):
                j = t * NBUF + b
                gathers[2 * b].wait()
                gathers[2 * b + 1].wait()
                pltpu.async_copy(vals0[b], acc0_sh.at[didx.at[j]], ssem[b],
                                 add=True)
                pltpu.async_copy(vals1[b], acc1_sh.at[didx.at[j]], ssem[b],
                                 add=True)
            return carry

        lax.fori_loop(0, CW // NBUF, step, 0)
        for b in range(NBUF):
            pltpu.make_async_copy(vals0[b], acc0_sh.at[didx.at[0]], ssem[b]).wait()
            pltpu.make_async_copy(vals1[b], acc1_sh.at[didx.at[0]], ssem[b]).wait()
        plsc.subcore_barrier()
        pltpu.sync_copy(acc0_sh.at[sp], out_hbm.at[c, 0, sp])
        pltpu.sync_copy(acc1_sh.at[sp], out_hbm.at[c, 1, sp])

    return thin_prop


def _deg_to_dinv(deg_ref, dd_ref):
    d = deg_ref[...]
    deg = d[0:1, :] + d[1:2, :] + 1.0
    dinv = lax.rsqrt(deg)
    dd_ref[0:1, :] = dinv
    dd_ref[1:2, :] = dinv * dinv


def _scale_rows(x_ref, dinv_ref, u_ref):
    u_ref[...] = x_ref[...] * dinv_ref[...]


def _dense_block(acc0_ref, acc1_ref, x_ref, dinv_ref, dinv2_ref,
                 w1_ref, b1_ref, w2_ref, g_ref, u2_ref):
    y = dinv_ref[...] * (acc0_ref[...] + acc1_ref[...]) + dinv2_ref[...] * x_ref[...]
    h = jnp.dot(y, w1_ref[...], preferred_element_type=jnp.float32) + b1_ref[...]
    h = jnp.maximum(h, 0.0)
    g = jnp.dot(h, w2_ref[...], preferred_element_type=jnp.float32)
    g_ref[...] = g
    u2_ref[...] = dinv_ref[...] * g


def _make_final(NP, NB):
    def final_body(a20_ref, a21_ref, gt_ref, dd_ref, b2_ref, batch_ref, out_ref):
        dinv = dd_ref[0:1, :]
        dinv2 = dd_ref[1:2, :]
        z0 = (dinv * (a20_ref[0:1, :] + a20_ref[1:2, :])
              + dinv2 * gt_ref[0:1, :] + b2_ref[0, 0])
        z1 = (dinv * (a21_ref[0:1, :] + a21_ref[1:2, :])
              + dinv2 * gt_ref[1:2, :] + b2_ref[0, 1])
        bt = batch_ref[...]
        gids = lax.broadcasted_iota(jnp.int32, (NB, NP), 0)
        onehot = (bt == gids).astype(jnp.float32)
        cnt = jnp.sum(onehot, axis=1, keepdims=True)
        s0 = jnp.sum(onehot * z0, axis=1, keepdims=True)
        s1 = jnp.sum(onehot * z1, axis=1, keepdims=True)
        out_ref[...] = jnp.concatenate([s0, s1], axis=1) / jnp.maximum(cnt, 1.0)

    return final_body


def kernel(x, edge_index, batch, W1, b1, W2, b2):
    N, D_IN = x.shape
    E = edge_index.shape[1]
    D_H = W1.shape[1]
    D_OUT = W2.shape[1]
    NB = 16

    NP = ((N + 511) // 512) * 512
    CW = -(-E // (NC * NS * K))
    CWQ = max(8, NBUF)
    CW = ((CW + CWQ - 1) // CWQ) * CWQ
    E_pad = NC * NS * CW * K

    src2d = jnp.concatenate(
        [edge_index[0], jnp.full((E_pad - E,), N, jnp.int32)]).reshape(-1, K)
    dst2d = jnp.concatenate(
        [edge_index[1], jnp.full((E_pad - E,), N, jnp.int32)]).reshape(-1, K)
    xp = jnp.pad(x, ((0, NP - N), (0, 0)))
    batchp = jnp.pad(batch, (0, NP - N), constant_values=NB).reshape(1, NP)

    deg_p = _make_deg_kernel(CW, NP)(dst2d)

    dd = pl.pallas_call(
        _deg_to_dinv,
        out_shape=jax.ShapeDtypeStruct((2, NP), jnp.float32),
    )(deg_p)
    dinv_c = dd[0].reshape(NP, 1)
    dinv2_c = dd[1].reshape(NP, 1)

    RB = 1024
    nblk = NP // RB
    u = pl.pallas_call(
        _scale_rows,
        grid=(nblk,),
        in_specs=[
            pl.BlockSpec((RB, D_IN), lambda i: (i, 0)),
            pl.BlockSpec((RB, 1), lambda i: (i, 0)),
        ],
        out_specs=pl.BlockSpec((RB, D_IN), lambda i: (i, 0)),
        out_shape=jax.ShapeDtypeStruct((NP, D_IN), jnp.float32),
    )(xp, dinv_c)

    zeros_wide = jnp.zeros((NP, D_IN), jnp.float32)
    acc = _make_prop_kernel(CW, NP, D_IN)(src2d, dst2d, u, zeros_wide)

    g, u2 = pl.pallas_call(
        _dense_block,
        grid=(nblk,),
        in_specs=[
            pl.BlockSpec((RB, D_IN), lambda i: (i, 0)),
            pl.BlockSpec((RB, D_IN), lambda i: (i, 0)),
            pl.BlockSpec((RB, D_IN), lambda i: (i, 0)),
            pl.BlockSpec((RB, 1), lambda i: (i, 0)),
            pl.BlockSpec((RB, 1), lambda i: (i, 0)),
            pl.BlockSpec((D_IN, D_H), lambda i: (0, 0)),
            pl.BlockSpec((1, D_H), lambda i: (0, 0)),
            pl.BlockSpec((D_H, D_OUT), lambda i: (0, 0)),
        ],
        out_specs=[
            pl.BlockSpec((RB, D_OUT), lambda i: (i, 0)),
            pl.BlockSpec((RB, D_OUT), lambda i: (i, 0)),
        ],
        out_shape=[
            jax.ShapeDtypeStruct((NP, D_OUT), jnp.float32),
            jax.ShapeDtypeStruct((NP, D_OUT), jnp.float32),
        ],
    )(acc[0], acc[1], xp, dinv_c, dinv2_c, W1, b1.reshape(1, D_H), W2)

    gt = g.T
    u2t = u2.T
    acc2 = _make_thin_prop_kernel(CW, NP)(src2d, dst2d, u2t[0], u2t[1])

    out = pl.pallas_call(
        _make_final(NP, NB),
        out_shape=jax.ShapeDtypeStruct((NB, D_OUT), jnp.float32),
    )(acc2[:, 0, :], acc2[:, 1, :], gt, dd, b2.reshape(1, D_OUT), batchp)
    return out


# speedup vs baseline: 29.9832x; 1.9934x over previous
import functools

import jax
import jax.numpy as jnp
from jax import lax
from jax.experimental import pallas as pl
from jax.experimental.pallas import tpu as pltpu
from jax.experimental.pallas import tpu_sc as plsc

NC = 2
NS = 16
K = 128
NBUF = 1


def _mesh():
    return plsc.VectorSubcoreMesh(core_axis_name="c", subcore_axis_name="s")


def _make_deg_kernel(CW, NP):
    span = NP // NS

    @functools.partial(
        pl.kernel,
        mesh=_mesh(),
        out_type=jax.ShapeDtypeStruct((NC, NP), jnp.float32),
        scratch_types=[
            pltpu.VMEM((CW, K), jnp.int32),
            pltpu.VMEM((K,), jnp.float32),
            pltpu.VMEM((span,), jnp.float32),
            pltpu.VMEM_SHARED((NP,), jnp.float32),
            pltpu.SemaphoreType.DMA,
        ],
    )
    def deg_kernel(dst2d_hbm, out_hbm, didx, ones_v, zbuf, deg_sh, sem):
        c = lax.axis_index("c")
        s = lax.axis_index("s")
        w = c * NS + s
        for i in range(K // 16):
            ones_v[pl.ds(16 * i, 16)] = jnp.ones((16,), jnp.float32)
        for i in range(span // 16):
            zbuf[pl.ds(16 * i, 16)] = jnp.zeros((16,), jnp.float32)
        pltpu.sync_copy(dst2d_hbm.at[pl.ds(w * CW, CW), :], didx)
        pltpu.sync_copy(zbuf, deg_sh.at[pl.ds(s * span, span)])
        plsc.subcore_barrier()

        def fire(j, carry):
            pltpu.async_copy(ones_v, deg_sh.at[didx.at[j]], sem, add=True)
            return carry

        def drain(j, carry):
            pltpu.make_async_copy(ones_v, deg_sh.at[didx.at[j]], sem).wait()
            return carry

        lax.fori_loop(0, CW, fire, 0)
        lax.fori_loop(0, CW, drain, 0)
        plsc.subcore_barrier()
        pltpu.sync_copy(deg_sh.at[pl.ds(s * span, span)],
                        out_hbm.at[c, pl.ds(s * span, span)])

    return deg_kernel


def _make_prop_kernel(CW, NP, D):
    span = NP // NS
    assert CW % NBUF == 0

    @functools.partial(
        pl.kernel,
        mesh=_mesh(),
        out_type=jax.ShapeDtypeStruct((NC, NP, D), jnp.float32),
        scratch_types=(
            [pltpu.VMEM((CW, K), jnp.int32),
             pltpu.VMEM((CW, K), jnp.int32)]
            + [pltpu.VMEM((K, D), jnp.float32)] * NBUF
            + [pltpu.VMEM_SHARED((NP, D), jnp.float32)]
            + [pltpu.SemaphoreType.DMA] * (2 * NBUF)
        ),
    )
    def prop_kernel(src2d_hbm, dst2d_hbm, u_hbm, zeros_hbm, out_hbm,
                    sidx, didx, *rest):
        rows = rest[:NBUF]
        acc_sh = rest[NBUF]
        gsem = rest[NBUF + 1:NBUF + 1 + NBUF]
        ssem = rest[NBUF + 1 + NBUF:]
        c = lax.axis_index("c")
        s = lax.axis_index("s")
        w = c * NS + s
        sp = pl.ds(s * span, span)
        pltpu.sync_copy(src2d_hbm.at[pl.ds(w * CW, CW), :], sidx)
        pltpu.sync_copy(dst2d_hbm.at[pl.ds(w * CW, CW), :], didx)
        pltpu.sync_copy(zeros_hbm.at[sp, :], acc_sh.at[sp, :])
        plsc.subcore_barrier()

        for b in range(NBUF):
            pltpu.async_copy(u_hbm.at[sidx.at[b]], rows[b], gsem[b])

        def step(t, carry):
            for b in range(NBUF):
                j = t * NBUF + b
                pltpu.make_async_copy(
                    u_hbm.at[sidx.at[j]], rows[b], gsem[b]).wait()
                pltpu.sync_copy(rows[b], acc_sh.at[didx.at[j]], add=True)

                @pl.when(j + NBUF < CW)
                def _():
                    pltpu.async_copy(
                        u_hbm.at[sidx.at[j + NBUF]], rows[b], gsem[b])
            return carry

        lax.fori_loop(0, CW // NBUF, step, 0)
        plsc.subcore_barrier()
        pltpu.sync_copy(acc_sh.at[sp, :], out_hbm.at[c, sp, :])

    return prop_kernel


def _make_thin_prop_kernel(CW, NP):
    span = NP // NS
    assert CW % NBUF == 0

    @functools.partial(
        pl.kernel,
        mesh=_mesh(),
        out_type=jax.ShapeDtypeStruct((NC, 2, NP), jnp.float32),
        scratch_types=(
            [pltpu.VMEM((CW, K), jnp.int32),
             pltpu.VMEM((CW, K), jnp.int32)]
            + [pltpu.VMEM((K,), jnp.float32)] * (2 * NBUF)
            + [pltpu.VMEM((span,), jnp.float32)]
            + [pltpu.VMEM_SHARED((NP,), jnp.float32)] * 2
            + [pltpu.SemaphoreType.DMA] * (2 * NBUF)
        ),
    )
    def thin_prop(src2d_hbm, dst2d_hbm, u20_hbm, u21_hbm, out_hbm,
                  sidx, didx, *rest):
        vals0 = rest[:NBUF]
        vals1 = rest[NBUF:2 * NBUF]
        zbuf = rest[2 * NBUF]
        acc0_sh = rest[2 * NBUF + 1]
        acc1_sh = rest[2 * NBUF + 2]
        gsem = rest[2 * NBUF + 3:2 * NBUF + 3 + NBUF]
        ssem = rest[2 * NBUF + 3 + NBUF:]
        c = lax.axis_index("c")
        s = lax.axis_index("s")
        w = c * NS + s
        sp = pl.ds(s * span, span)
        pltpu.sync_copy(src2d_hbm.at[pl.ds(w * CW, CW), :], sidx)
        pltpu.sync_copy(dst2d_hbm.at[pl.ds(w * CW, CW), :], didx)
        for i in range(span // 16):
            zbuf[pl.ds(16 * i, 16)] = jnp.zeros((16,), jnp.float32)
        pltpu.sync_copy(zbuf, acc0_sh.at[sp])
        pltpu.sync_copy(zbuf, acc1_sh.at[sp])
        plsc.subcore_barrier()

        def step(t, carry):
            @pl.when(t > 0)
            def _():
                for b in range(NBUF):
                    pltpu.make_async_copy(
                        vals0[b], acc0_sh.at[didx.at[0]], ssem[b]).wait()
                    pltpu.make_async_copy(
                        vals1[b], acc1_sh.at[didx.at[0]], ssem[b]).wait()

            gathers = []
            for b in range(NBUF):
                j = t * NBUF + b
                gathers.append(
                    pltpu.async_copy(u20_hbm.at[sidx.at[j]], vals0[b], gsem[b]))
                gathers.append(
                    pltpu.async_copy(u21_hbm.at[sidx.at[j]], vals1[b], gsem[b]))
            for b in range(NBUF):
                j = t * NBUF + b
                gathers[2 * b].wait()
                gathers[2 * b + 1].wait()
                pltpu.async_copy(vals0[b], acc0_sh.at[didx.at[j]], ssem[b],
                                 add=True)
                pltpu.async_copy(vals1[b], acc1_sh.at[didx.at[j]], ssem[b],
                                 add=True)
            return carry

        lax.fori_loop(0, CW // NBUF, step, 0)
        for b in range(NBUF):
            pltpu.make_async_copy(vals0[b], acc0_sh.at[didx.at[0]], ssem[b]).wait()
            pltpu.make_async_copy(vals1[b], acc1_sh.at[didx.at[0]], ssem[b]).wait()
        plsc.subcore_barrier()
        pltpu.sync_copy(acc0_sh.at[sp], out_hbm.at[c, 0, sp])
        pltpu.sync_copy(acc1_sh.at[sp], out_hbm.at[c, 1, sp])

    return thin_prop


def _deg_to_dinv(deg_ref, dd_ref):
    d = deg_ref[...]
    deg = d[0:1, :] + d[1:2, :] + 1.0
    dinv = lax.rsqrt(deg)
    dd_ref[0:1, :] = dinv
    dd_ref[1:2, :] = dinv * dinv


def _scale_rows(x_ref, dinv_ref, u_ref):
    u_ref[...] = x_ref[...] * dinv_ref[...]


def _dense_block(acc0_ref, acc1_ref, x_ref, dinv_ref, dinv2_ref,
                 w1_ref, b1_ref, w2_ref, g_ref, u2_ref):
    y = dinv_ref[...] * (acc0_ref[...] + acc1_ref[...]) + dinv2_ref[...] * x_ref[...]
    h = jnp.dot(y, w1_ref[...], preferred_element_type=jnp.float32) + b1_ref[...]
    h = jnp.maximum(h, 0.0)
    g = jnp.dot(h, w2_ref[...], preferred_element_type=jnp.float32)
    g_ref[...] = g
    u2_ref[...] = dinv_ref[...] * g


def _make_final(NP, NB):
    def final_body(a20_ref, a21_ref, gt_ref, dd_ref, b2_ref, batch_ref, out_ref):
        dinv = dd_ref[0:1, :]
        dinv2 = dd_ref[1:2, :]
        z0 = (dinv * (a20_ref[0:1, :] + a20_ref[1:2, :])
              + dinv2 * gt_ref[0:1, :] + b2_ref[0, 0])
        z1 = (dinv * (a21_ref[0:1, :] + a21_ref[1:2, :])
              + dinv2 * gt_ref[1:2, :] + b2_ref[0, 1])
        bt = batch_ref[...]
        gids = lax.broadcasted_iota(jnp.int32, (NB, NP), 0)
        onehot = (bt == gids).astype(jnp.float32)
        cnt = jnp.sum(onehot, axis=1, keepdims=True)
        s0 = jnp.sum(onehot * z0, axis=1, keepdims=True)
        s1 = jnp.sum(onehot * z1, axis=1, keepdims=True)
        out_ref[...] = jnp.concatenate([s0, s1], axis=1) / jnp.maximum(cnt, 1.0)

    return final_body


def kernel(x, edge_index, batch, W1, b1, W2, b2):
    N, D_IN = x.shape
    E = edge_index.shape[1]
    D_H = W1.shape[1]
    D_OUT = W2.shape[1]
    NB = 16

    NP = ((N + 511) // 512) * 512
    CW = -(-E // (NC * NS * K))
    CWQ = max(8, NBUF)
    CW = ((CW + CWQ - 1) // CWQ) * CWQ
    E_pad = NC * NS * CW * K

    pad_ids = N + jnp.arange(E_pad - E, dtype=jnp.int32) % (NP - N)
    src2d = jnp.concatenate([edge_index[0], pad_ids]).reshape(-1, K)
    dst2d = jnp.concatenate([edge_index[1], pad_ids]).reshape(-1, K)
    xp = jnp.pad(x, ((0, NP - N), (0, 0)))
    batchp = jnp.pad(batch, (0, NP - N), constant_values=NB).reshape(1, NP)

    deg_p = _make_deg_kernel(CW, NP)(dst2d)

    dd = pl.pallas_call(
        _deg_to_dinv,
        out_shape=jax.ShapeDtypeStruct((2, NP), jnp.float32),
    )(deg_p)
    dinv_c = dd[0].reshape(NP, 1)
    dinv2_c = dd[1].reshape(NP, 1)

    RB = 1024
    nblk = NP // RB
    u = pl.pallas_call(
        _scale_rows,
        grid=(nblk,),
        in_specs=[
            pl.BlockSpec((RB, D_IN), lambda i: (i, 0)),
            pl.BlockSpec((RB, 1), lambda i: (i, 0)),
        ],
        out_specs=pl.BlockSpec((RB, D_IN), lambda i: (i, 0)),
        out_shape=jax.ShapeDtypeStruct((NP, D_IN), jnp.float32),
    )(xp, dinv_c)

    zeros_wide = jnp.zeros((NP, D_IN), jnp.float32)
    acc = _make_prop_kernel(CW, NP, D_IN)(src2d, dst2d, u, zeros_wide)

    g, u2 = pl.pallas_call(
        _dense_block,
        grid=(nblk,),
        in_specs=[
            pl.BlockSpec((RB, D_IN), lambda i: (i, 0)),
            pl.BlockSpec((RB, D_IN), lambda i: (i, 0)),
            pl.BlockSpec((RB, D_IN), lambda i: (i, 0)),
            pl.BlockSpec((RB, 1), lambda i: (i, 0)),
            pl.BlockSpec((RB, 1), lambda i: (i, 0)),
            pl.BlockSpec((D_IN, D_H), lambda i: (0, 0)),
            pl.BlockSpec((1, D_H), lambda i: (0, 0)),
            pl.BlockSpec((D_H, D_OUT), lambda i: (0, 0)),
        ],
        out_specs=[
            pl.BlockSpec((RB, D_OUT), lambda i: (i, 0)),
            pl.BlockSpec((RB, D_OUT), lambda i: (i, 0)),
        ],
        out_shape=[
            jax.ShapeDtypeStruct((NP, D_OUT), jnp.float32),
            jax.ShapeDtypeStruct((NP, D_OUT), jnp.float32),
        ],
    )(acc[0], acc[1], xp, dinv_c, dinv2_c, W1, b1.reshape(1, D_H), W2)

    gt = g.T
    u2t = u2.T
    acc2 = _make_thin_prop_kernel(CW, NP)(src2d, dst2d, u2t[0], u2t[1])

    out = pl.pallas_call(
        _make_final(NP, NB),
        out_shape=jax.ShapeDtypeStruct((NB, D_OUT), jnp.float32),
    )(acc2[:, 0, :], acc2[:, 1, :], gt, dd, b2.reshape(1, D_OUT), batchp)
    return out


# speedup vs baseline: 36.5601x; 1.2194x over previous
import functools

import jax
import jax.numpy as jnp
from jax import lax
from jax.experimental import pallas as pl
from jax.experimental.pallas import tpu as pltpu
from jax.experimental.pallas import tpu_sc as plsc

NC = 2
NS = 16
K = 128
NBUF = 1


def _mesh():
    return plsc.VectorSubcoreMesh(core_axis_name="c", subcore_axis_name="s")


def _make_deg_kernel(CW, NP):
    span = NP // NS

    @functools.partial(
        pl.kernel,
        mesh=_mesh(),
        out_type=jax.ShapeDtypeStruct((NC, NP), jnp.float32),
        scratch_types=[
            pltpu.VMEM((CW, K), jnp.int32),
            pltpu.VMEM((K,), jnp.float32),
            pltpu.VMEM((span,), jnp.float32),
            pltpu.VMEM_SHARED((NP,), jnp.float32),
            pltpu.SemaphoreType.DMA,
        ],
    )
    def deg_kernel(dst2d_hbm, out_hbm, didx, ones_v, zbuf, deg_sh, sem):
        c = lax.axis_index("c")
        s = lax.axis_index("s")
        w = c * NS + s
        for i in range(K // 16):
            ones_v[pl.ds(16 * i, 16)] = jnp.ones((16,), jnp.float32)
        for i in range(span // 16):
            zbuf[pl.ds(16 * i, 16)] = jnp.zeros((16,), jnp.float32)
        pltpu.sync_copy(dst2d_hbm.at[pl.ds(w * CW, CW), :], didx)
        pltpu.sync_copy(zbuf, deg_sh.at[pl.ds(s * span, span)])
        plsc.subcore_barrier()

        def fire(j, carry):
            pltpu.async_copy(ones_v, deg_sh.at[didx.at[j]], sem, add=True)
            return carry

        def drain(j, carry):
            pltpu.make_async_copy(ones_v, deg_sh.at[didx.at[j]], sem).wait()
            return carry

        lax.fori_loop(0, CW, fire, 0)
        lax.fori_loop(0, CW, drain, 0)
        plsc.subcore_barrier()
        pltpu.sync_copy(deg_sh.at[pl.ds(s * span, span)],
                        out_hbm.at[c, pl.ds(s * span, span)])

    return deg_kernel


def _make_prop_kernel(CW, NP, D):
    span = NP // NS
    PH = 2
    CWp = CW // PH
    assert CW % (2 * PH) == 0 and CWp % 8 == 0

    @functools.partial(
        pl.kernel,
        mesh=_mesh(),
        out_type=jax.ShapeDtypeStruct((NC, NP, D), jnp.float32),
        scratch_types=(
            [pltpu.VMEM((CWp, K), jnp.int32),
             pltpu.VMEM((CWp, K), jnp.int32)]
            + [pltpu.VMEM((2 * K, D), jnp.float32)]
            + [pltpu.VMEM_SHARED((NP, D), jnp.float32)]
            + [pltpu.SemaphoreType.DMA] * 2
        ),
    )
    def prop_kernel(src2d_hbm, dst2d_hbm, u_hbm, zeros_hbm, out_hbm,
                    sidx, didx, rows_buf, acc_sh, *gsem):
        rows = [rows_buf.at[pl.ds(b * K, K), :] for b in range(2)]
        c = lax.axis_index("c")
        s = lax.axis_index("s")
        w = c * NS + s
        sp = pl.ds(s * span, span)
        pltpu.sync_copy(zeros_hbm.at[sp, :], acc_sh.at[sp, :])
        plsc.subcore_barrier()

        for p in range(PH):
            base = w * CW + p * CWp
            pltpu.sync_copy(src2d_hbm.at[pl.ds(base, CWp), :], sidx)
            pltpu.sync_copy(dst2d_hbm.at[pl.ds(base, CWp), :], didx)
            for b in range(2):
                pltpu.async_copy(u_hbm.at[sidx.at[b]], rows[b], gsem[b])

            def step(t, carry):
                for b in range(2):
                    j = t * 2 + b
                    pltpu.make_async_copy(
                        u_hbm.at[sidx.at[j]], rows[b], gsem[b]).wait()
                    pltpu.sync_copy(rows[b], acc_sh.at[didx.at[j]], add=True)

                    @pl.when(j + 2 < CWp)
                    def _():
                        pltpu.async_copy(
                            u_hbm.at[sidx.at[j + 2]], rows[b], gsem[b])
                return carry

            lax.fori_loop(0, CWp // 2, step, 0)

        plsc.subcore_barrier()
        pltpu.sync_copy(acc_sh.at[sp, :], out_hbm.at[c, sp, :])

    return prop_kernel


def _make_thin_prop_kernel(CW, NP):
    span = NP // NS
    assert CW % NBUF == 0

    @functools.partial(
        pl.kernel,
        mesh=_mesh(),
        out_type=jax.ShapeDtypeStruct((NC, 2, NP), jnp.float32),
        scratch_types=(
            [pltpu.VMEM((CW, K), jnp.int32),
             pltpu.VMEM((CW, K), jnp.int32)]
            + [pltpu.VMEM((K,), jnp.float32)] * (2 * NBUF)
            + [pltpu.VMEM((span,), jnp.float32)]
            + [pltpu.VMEM_SHARED((NP,), jnp.float32)] * 2
            + [pltpu.SemaphoreType.DMA] * (2 * NBUF)
        ),
    )
    def thin_prop(src2d_hbm, dst2d_hbm, u20_hbm, u21_hbm, out_hbm,
                  sidx, didx, *rest):
        vals0 = rest[:NBUF]
        vals1 = rest[NBUF:2 * NBUF]
        zbuf = rest[2 * NBUF]
        acc0_sh = rest[2 * NBUF + 1]
        acc1_sh = rest[2 * NBUF + 2]
        gsem = rest[2 * NBUF + 3:2 * NBUF + 3 + NBUF]
        ssem = rest[2 * NBUF + 3 + NBUF:]
        c = lax.axis_index("c")
        s = lax.axis_index("s")
        w = c * NS + s
        sp = pl.ds(s * span, span)
        pltpu.sync_copy(src2d_hbm.at[pl.ds(w * CW, CW), :], sidx)
        pltpu.sync_copy(dst2d_hbm.at[pl.ds(w * CW, CW), :], didx)
        for i in range(span // 16):
            zbuf[pl.ds(16 * i, 16)] = jnp.zeros((16,), jnp.float32)
        pltpu.sync_copy(zbuf, acc0_sh.at[sp])
        pltpu.sync_copy(zbuf, acc1_sh.at[sp])
        plsc.subcore_barrier()

        def step(t, carry):
            @pl.when(t > 0)
            def _():
                for b in range(NBUF):
                    pltpu.make_async_copy(
                        vals0[b], acc0_sh.at[didx.at[0]], ssem[b]).wait()
                    pltpu.make_async_copy(
                        vals1[b], acc1_sh.at[didx.at[0]], ssem[b]).wait()

            gathers = []
            for b in range(NBUF):
                j = t * NBUF + b
                gathers.append(
                    pltpu.async_copy(u20_hbm.at[sidx.at[j]], vals0[b], gsem[b]))
                gathers.append(
                    pltpu.async_copy(u21_hbm.at[sidx.at[j]], vals1[b], gsem[b]))
            for b in range(NBUF):
                j = t * NBUF + b
                gathers[2 * b].wait()
                gathers[2 * b + 1].wait()
                pltpu.async_copy(vals0[b], acc0_sh.at[didx.at[j]], ssem[b],
                                 add=True)
                pltpu.async_copy(vals1[b], acc1_sh.at[didx.at[j]], ssem[b],
                                 add=True)
            return carry

        lax.fori_loop(0, CW // NBUF, step, 0)
        for b in range(NBUF):
            pltpu.make_async_copy(vals0[b], acc0_sh.at[didx.at[0]], ssem[b]).wait()
            pltpu.make_async_copy(vals1[b], acc1_sh.at[didx.at[0]], ssem[b]).wait()
        plsc.subcore_barrier()
        pltpu.sync_copy(acc0_sh.at[sp], out_hbm.at[c, 0, sp])
        pltpu.sync_copy(acc1_sh.at[sp], out_hbm.at[c, 1, sp])

    return thin_prop


def _deg_to_dinv(deg_ref, dd_ref):
    d = deg_ref[...]
    deg = d[0:1, :] + d[1:2, :] + 1.0
    dinv = lax.rsqrt(deg)
    dd_ref[0:1, :] = dinv
    dd_ref[1:2, :] = dinv * dinv


def _scale_rows(x_ref, dinv_ref, u_ref):
    u_ref[...] = x_ref[...] * dinv_ref[...]


def _dense_block(acc0_ref, acc1_ref, x_ref, dinv_ref, dinv2_ref,
                 w1_ref, b1_ref, w2_ref, g_ref, u2_ref):
    y = dinv_ref[...] * (acc0_ref[...] + acc1_ref[...]) + dinv2_ref[...] * x_ref[...]
    h = jnp.dot(y, w1_ref[...], preferred_element_type=jnp.float32) + b1_ref[...]
    h = jnp.maximum(h, 0.0)
    g = jnp.dot(h, w2_ref[...], preferred_element_type=jnp.float32)
    g_ref[...] = g
    u2_ref[...] = dinv_ref[...] * g


def _make_final(NP, NB):
    def final_body(a20_ref, a21_ref, gt_ref, dd_ref, b2_ref, batch_ref, out_ref):
        dinv = dd_ref[0:1, :]
        dinv2 = dd_ref[1:2, :]
        z0 = (dinv * (a20_ref[0:1, :] + a20_ref[1:2, :])
              + dinv2 * gt_ref[0:1, :] + b2_ref[0, 0])
        z1 = (dinv * (a21_ref[0:1, :] + a21_ref[1:2, :])
              + dinv2 * gt_ref[1:2, :] + b2_ref[0, 1])
        bt = batch_ref[...]
        gids = lax.broadcasted_iota(jnp.int32, (NB, NP), 0)
        onehot = (bt == gids).astype(jnp.float32)
        cnt = jnp.sum(onehot, axis=1, keepdims=True)
        s0 = jnp.sum(onehot * z0, axis=1, keepdims=True)
        s1 = jnp.sum(onehot * z1, axis=1, keepdims=True)
        out_ref[...] = jnp.concatenate([s0, s1], axis=1) / jnp.maximum(cnt, 1.0)

    return final_body


def kernel(x, edge_index, batch, W1, b1, W2, b2):
    N, D_IN = x.shape
    E = edge_index.shape[1]
    D_H = W1.shape[1]
    D_OUT = W2.shape[1]
    NB = 16

    NP = ((N + 511) // 512) * 512
    CW = -(-E // (NC * NS * K))
    CWQ = max(8, NBUF)
    CW = ((CW + CWQ - 1) // CWQ) * CWQ
    E_pad = NC * NS * CW * K

    pad_src = N + jnp.arange(E_pad - E + 8 * K, dtype=jnp.int32) % (NP - N)
    pad_dst = pad_src[:E_pad - E]
    src2d = jnp.concatenate([edge_index[0], pad_src]).reshape(-1, K)
    dst2d = jnp.concatenate([edge_index[1], pad_dst]).reshape(-1, K)
    xp = jnp.pad(x, ((0, NP - N), (0, 0)))
    batchp = jnp.pad(batch, (0, NP - N), constant_values=NB).reshape(1, NP)

    deg_p = _make_deg_kernel(CW, NP)(dst2d)

    dd = pl.pallas_call(
        _deg_to_dinv,
        out_shape=jax.ShapeDtypeStruct((2, NP), jnp.float32),
    )(deg_p)
    dinv_c = dd[0].reshape(NP, 1)
    dinv2_c = dd[1].reshape(NP, 1)

    RB = 1024
    nblk = NP // RB
    u = pl.pallas_call(
        _scale_rows,
        grid=(nblk,),
        in_specs=[
            pl.BlockSpec((RB, D_IN), lambda i: (i, 0)),
            pl.BlockSpec((RB, 1), lambda i: (i, 0)),
        ],
        out_specs=pl.BlockSpec((RB, D_IN), lambda i: (i, 0)),
        out_shape=jax.ShapeDtypeStruct((NP, D_IN), jnp.float32),
    )(xp, dinv_c)

    zeros_wide = jnp.zeros((NP, D_IN), jnp.float32)
    acc = _make_prop_kernel(CW, NP, D_IN)(src2d, dst2d, u, zeros_wide)

    g, u2 = pl.pallas_call(
        _dense_block,
        grid=(nblk,),
        in_specs=[
            pl.BlockSpec((RB, D_IN), lambda i: (i, 0)),
            pl.BlockSpec((RB, D_IN), lambda i: (i, 0)),
            pl.BlockSpec((RB, D_IN), lambda i: (i, 0)),
            pl.BlockSpec((RB, 1), lambda i: (i, 0)),
            pl.BlockSpec((RB, 1), lambda i: (i, 0)),
            pl.BlockSpec((D_IN, D_H), lambda i: (0, 0)),
            pl.BlockSpec((1, D_H), lambda i: (0, 0)),
            pl.BlockSpec((D_H, D_OUT), lambda i: (0, 0)),
        ],
        out_specs=[
            pl.BlockSpec((RB, D_OUT), lambda i: (i, 0)),
            pl.BlockSpec((RB, D_OUT), lambda i: (i, 0)),
        ],
        out_shape=[
            jax.ShapeDtypeStruct((NP, D_OUT), jnp.float32),
            jax.ShapeDtypeStruct((NP, D_OUT), jnp.float32),
        ],
    )(acc[0], acc[1], xp, dinv_c, dinv2_c, W1, b1.reshape(1, D_H), W2)

    gt = g.T
    u2t = u2.T
    acc2 = _make_thin_prop_kernel(CW, NP)(src2d, dst2d, u2t[0], u2t[1])

    out = pl.pallas_call(
        _make_final(NP, NB),
        out_shape=jax.ShapeDtypeStruct((NB, D_OUT), jnp.float32),
    )(acc2[:, 0, :], acc2[:, 1, :], gt, dd, b2.reshape(1, D_OUT), batchp)
    return out


# speedup vs baseline: 42.4941x; 1.1623x over previous
import functools

import jax
import jax.numpy as jnp
from jax import lax
from jax.experimental import pallas as pl
from jax.experimental.pallas import tpu as pltpu
from jax.experimental.pallas import tpu_sc as plsc

NC = 2
NS = 16
K = 128
NBUF = 1


def _mesh():
    return plsc.VectorSubcoreMesh(core_axis_name="c", subcore_axis_name="s")


def _make_deg_kernel(CW, NP):
    span = NP // NS

    @functools.partial(
        pl.kernel,
        mesh=_mesh(),
        out_type=jax.ShapeDtypeStruct((NC, NP), jnp.float32),
        scratch_types=[
            pltpu.VMEM((CW, K), jnp.int32),
            pltpu.VMEM((K,), jnp.float32),
            pltpu.VMEM((span,), jnp.float32),
            pltpu.VMEM_SHARED((NP,), jnp.float32),
            pltpu.SemaphoreType.DMA,
        ],
    )
    def deg_kernel(dst2d_hbm, out_hbm, didx, ones_v, zbuf, deg_sh, sem):
        c = lax.axis_index("c")
        s = lax.axis_index("s")
        w = c * NS + s
        for i in range(K // 16):
            ones_v[pl.ds(16 * i, 16)] = jnp.ones((16,), jnp.float32)
        for i in range(span // 16):
            zbuf[pl.ds(16 * i, 16)] = jnp.zeros((16,), jnp.float32)
        pltpu.sync_copy(dst2d_hbm.at[pl.ds(w * CW, CW), :], didx)
        pltpu.sync_copy(zbuf, deg_sh.at[pl.ds(s * span, span)])
        plsc.subcore_barrier()

        def fire(j, carry):
            pltpu.async_copy(ones_v, deg_sh.at[didx.at[j]], sem, add=True)
            return carry

        def drain(j, carry):
            pltpu.make_async_copy(ones_v, deg_sh.at[didx.at[j]], sem).wait()
            return carry

        lax.fori_loop(0, CW, fire, 0)
        lax.fori_loop(0, CW, drain, 0)
        plsc.subcore_barrier()
        pltpu.sync_copy(deg_sh.at[pl.ds(s * span, span)],
                        out_hbm.at[c, pl.ds(s * span, span)])

    return deg_kernel


def _make_prop_kernel(CW, NP, D):
    span = NP // NS
    PH = 2
    CWp = CW // PH
    assert CW % (2 * PH) == 0 and CWp % 8 == 0

    @functools.partial(
        pl.kernel,
        mesh=_mesh(),
        out_type=jax.ShapeDtypeStruct((NC, NP, D), jnp.float32),
        scratch_types=(
            [pltpu.VMEM((CWp, K), jnp.int32),
             pltpu.VMEM((CWp, K), jnp.int32)]
            + [pltpu.VMEM((2 * K, D), jnp.float32)]
            + [pltpu.VMEM_SHARED((NP, D), jnp.float32)]
            + [pltpu.SemaphoreType.DMA] * 2
        ),
    )
    def prop_kernel(src2d_hbm, dst2d_hbm, u_hbm, zeros_hbm, out_hbm,
                    sidx, didx, rows_buf, acc_sh, *gsem):
        rows = [rows_buf.at[pl.ds(b * K, K), :] for b in range(2)]
        c = lax.axis_index("c")
        s = lax.axis_index("s")
        w = c * NS + s
        sp = pl.ds(s * span, span)
        pltpu.sync_copy(zeros_hbm.at[sp, :], acc_sh.at[sp, :])
        plsc.subcore_barrier()

        for p in range(PH):
            base = w * CW + p * CWp
            pltpu.sync_copy(src2d_hbm.at[pl.ds(base, CWp), :], sidx)
            pltpu.sync_copy(dst2d_hbm.at[pl.ds(base, CWp), :], didx)
            for b in range(2):
                pltpu.async_copy(u_hbm.at[sidx.at[b]], rows[b], gsem[b])

            def step(t, carry):
                for b in range(2):
                    j = t * 2 + b
                    pltpu.make_async_copy(
                        u_hbm.at[sidx.at[j]], rows[b], gsem[b]).wait()
                    pltpu.sync_copy(rows[b], acc_sh.at[didx.at[j]], add=True)

                    @pl.when(j + 2 < CWp)
                    def _():
                        pltpu.async_copy(
                            u_hbm.at[sidx.at[j + 2]], rows[b], gsem[b])
                return carry

            lax.fori_loop(0, CWp // 2, step, 0)

        plsc.subcore_barrier()
        pltpu.sync_copy(acc_sh.at[sp, :], out_hbm.at[c, sp, :])

    return prop_kernel


def _make_thin_prop_kernel(CW, NP):
    span = NP // NS
    assert CW % 2 == 0

    @functools.partial(
        pl.kernel,
        mesh=_mesh(),
        out_type=jax.ShapeDtypeStruct((NC, 2, NP), jnp.float32),
        scratch_types=(
            [pltpu.VMEM((CW, K), jnp.int32),
             pltpu.VMEM((CW, K), jnp.int32)]
            + [pltpu.VMEM((K,), jnp.float32)] * 4
            + [pltpu.VMEM((span,), jnp.float32)]
            + [pltpu.VMEM_SHARED((NP,), jnp.float32)] * 2
            + [pltpu.SemaphoreType.DMA] * 2
        ),
    )
    def thin_prop(src2d_hbm, dst2d_hbm, u20_hbm, u21_hbm, out_hbm,
                  sidx, didx, *rest):
        vals0 = rest[0:2]
        vals1 = rest[2:4]
        zbuf = rest[4]
        acc0_sh = rest[5]
        acc1_sh = rest[6]
        gsem = rest[7:9]
        c = lax.axis_index("c")
        s = lax.axis_index("s")
        w = c * NS + s
        sp = pl.ds(s * span, span)
        pltpu.sync_copy(src2d_hbm.at[pl.ds(w * CW, CW), :], sidx)
        pltpu.sync_copy(dst2d_hbm.at[pl.ds(w * CW, CW), :], didx)
        for i in range(span // 16):
            zbuf[pl.ds(16 * i, 16)] = jnp.zeros((16,), jnp.float32)
        pltpu.sync_copy(zbuf, acc0_sh.at[sp])
        pltpu.sync_copy(zbuf, acc1_sh.at[sp])
        plsc.subcore_barrier()

        for b in range(2):
            pltpu.async_copy(u20_hbm.at[sidx.at[b]], vals0[b], gsem[b])
            pltpu.async_copy(u21_hbm.at[sidx.at[b]], vals1[b], gsem[b])

        def step(t, carry):
            for b in range(2):
                j = t * 2 + b
                pltpu.make_async_copy(
                    u20_hbm.at[sidx.at[j]], vals0[b], gsem[b]).wait()
                pltpu.make_async_copy(
                    u21_hbm.at[sidx.at[j]], vals1[b], gsem[b]).wait()
                pltpu.sync_copy(vals0[b], acc0_sh.at[didx.at[j]], add=True)
                pltpu.sync_copy(vals1[b], acc1_sh.at[didx.at[j]], add=True)

                @pl.when(j + 2 < CW)
                def _():
                    pltpu.async_copy(u20_hbm.at[sidx.at[j + 2]], vals0[b],
                                     gsem[b])
                    pltpu.async_copy(u21_hbm.at[sidx.at[j + 2]], vals1[b],
                                     gsem[b])
            return carry

        lax.fori_loop(0, CW // 2, step, 0)
        plsc.subcore_barrier()
        pltpu.sync_copy(acc0_sh.at[sp], out_hbm.at[c, 0, sp])
        pltpu.sync_copy(acc1_sh.at[sp], out_hbm.at[c, 1, sp])

    return thin_prop


def _deg_to_dinv(deg_ref, dd_ref):
    d = deg_ref[...]
    deg = d[0:1, :] + d[1:2, :] + 1.0
    dinv = lax.rsqrt(deg)
    dd_ref[0:1, :] = dinv
    dd_ref[1:2, :] = dinv * dinv


def _scale_rows(x_ref, dinv_ref, u_ref):
    u_ref[...] = x_ref[...] * dinv_ref[...]


def _dense_block(acc0_ref, acc1_ref, x_ref, dinv_ref, dinv2_ref,
                 w1_ref, b1_ref, w2_ref, g_ref, u2_ref):
    y = dinv_ref[...] * (acc0_ref[...] + acc1_ref[...]) + dinv2_ref[...] * x_ref[...]
    h = jnp.dot(y, w1_ref[...], preferred_element_type=jnp.float32) + b1_ref[...]
    h = jnp.maximum(h, 0.0)
    g = jnp.dot(h, w2_ref[...], preferred_element_type=jnp.float32)
    g_ref[...] = g
    u2_ref[...] = dinv_ref[...] * g


def _make_final(NP, NB):
    def final_body(a20_ref, a21_ref, gt_ref, dd_ref, b2_ref, batch_ref, out_ref):
        dinv = dd_ref[0:1, :]
        dinv2 = dd_ref[1:2, :]
        z0 = (dinv * (a20_ref[0:1, :] + a20_ref[1:2, :])
              + dinv2 * gt_ref[0:1, :] + b2_ref[0, 0])
        z1 = (dinv * (a21_ref[0:1, :] + a21_ref[1:2, :])
              + dinv2 * gt_ref[1:2, :] + b2_ref[0, 1])
        bt = batch_ref[...]
        gids = lax.broadcasted_iota(jnp.int32, (NB, NP), 0)
        onehot = (bt == gids).astype(jnp.float32)
        cnt = jnp.sum(onehot, axis=1, keepdims=True)
        s0 = jnp.sum(onehot * z0, axis=1, keepdims=True)
        s1 = jnp.sum(onehot * z1, axis=1, keepdims=True)
        out_ref[...] = jnp.concatenate([s0, s1], axis=1) / jnp.maximum(cnt, 1.0)

    return final_body


def kernel(x, edge_index, batch, W1, b1, W2, b2):
    N, D_IN = x.shape
    E = edge_index.shape[1]
    D_H = W1.shape[1]
    D_OUT = W2.shape[1]
    NB = 16

    NP = ((N + 511) // 512) * 512
    CW = -(-E // (NC * NS * K))
    CWQ = max(8, NBUF)
    CW = ((CW + CWQ - 1) // CWQ) * CWQ
    E_pad = NC * NS * CW * K

    pad_src = N + jnp.arange(E_pad - E + 8 * K, dtype=jnp.int32) % (NP - N)
    pad_dst = pad_src[:E_pad - E]
    src2d = jnp.concatenate([edge_index[0], pad_src]).reshape(-1, K)
    dst2d = jnp.concatenate([edge_index[1], pad_dst]).reshape(-1, K)
    xp = jnp.pad(x, ((0, NP - N), (0, 0)))
    batchp = jnp.pad(batch, (0, NP - N), constant_values=NB).reshape(1, NP)

    deg_p = _make_deg_kernel(CW, NP)(dst2d)

    dd = pl.pallas_call(
        _deg_to_dinv,
        out_shape=jax.ShapeDtypeStruct((2, NP), jnp.float32),
    )(deg_p)
    dinv_c = dd[0].reshape(NP, 1)
    dinv2_c = dd[1].reshape(NP, 1)

    RB = 1024
    nblk = NP // RB
    u = pl.pallas_call(
        _scale_rows,
        grid=(nblk,),
        in_specs=[
            pl.BlockSpec((RB, D_IN), lambda i: (i, 0)),
            pl.BlockSpec((RB, 1), lambda i: (i, 0)),
        ],
        out_specs=pl.BlockSpec((RB, D_IN), lambda i: (i, 0)),
        out_shape=jax.ShapeDtypeStruct((NP, D_IN), jnp.float32),
    )(xp, dinv_c)

    zeros_wide = jnp.zeros((NP, D_IN), jnp.float32)
    acc = _make_prop_kernel(CW, NP, D_IN)(src2d, dst2d, u, zeros_wide)

    g, u2 = pl.pallas_call(
        _dense_block,
        grid=(nblk,),
        in_specs=[
            pl.BlockSpec((RB, D_IN), lambda i: (i, 0)),
            pl.BlockSpec((RB, D_IN), lambda i: (i, 0)),
            pl.BlockSpec((RB, D_IN), lambda i: (i, 0)),
            pl.BlockSpec((RB, 1), lambda i: (i, 0)),
            pl.BlockSpec((RB, 1), lambda i: (i, 0)),
            pl.BlockSpec((D_IN, D_H), lambda i: (0, 0)),
            pl.BlockSpec((1, D_H), lambda i: (0, 0)),
            pl.BlockSpec((D_H, D_OUT), lambda i: (0, 0)),
        ],
        out_specs=[
            pl.BlockSpec((RB, D_OUT), lambda i: (i, 0)),
            pl.BlockSpec((RB, D_OUT), lambda i: (i, 0)),
        ],
        out_shape=[
            jax.ShapeDtypeStruct((NP, D_OUT), jnp.float32),
            jax.ShapeDtypeStruct((NP, D_OUT), jnp.float32),
        ],
    )(acc[0], acc[1], xp, dinv_c, dinv2_c, W1, b1.reshape(1, D_H), W2)

    gt = g.T
    u2t = u2.T
    acc2 = _make_thin_prop_kernel(CW, NP)(src2d, dst2d, u2t[0], u2t[1])

    out = pl.pallas_call(
        _make_final(NP, NB),
        out_shape=jax.ShapeDtypeStruct((NB, D_OUT), jnp.float32),
    )(acc2[:, 0, :], acc2[:, 1, :], gt, dd, b2.reshape(1, D_OUT), batchp)
    return out
